# baseline (speedup 1.0000x reference)
import jax
import jax.numpy as jnp
from jax.experimental import pallas as pl
from jax.experimental.pallas import tpu as pltpu

_VOCAB = 8192
_EMBED = 256
_BV = 512
_NBLK = _VOCAB // _BV
_HALF_MERGE = _VOCAB // (2 * _BV) + 2


def _body(p_ref, t_ref, o_ref, pn_ref, sim_ref, mblk_ref, w_ref,
          a_blk_ref, m_ref, a_ref):
    v = pl.program_id(0)
    nsteps = pl.num_programs(0)

    @pl.when(v == 0)
    def _():
        p = p_ref[...]
        pnorm = jnp.sqrt(jnp.sum(p * p, axis=1, keepdims=True))
        pn_ref[...] = (p / jnp.maximum(pnorm, 1e-12)).astype(jnp.bfloat16)
        s = jax.lax.broadcasted_iota(jnp.int32, (1, _BV), 1)
        w_ref[0:1, :] = (s // 16).astype(jnp.float32)
        w_ref[1:2, :] = (s % 16).astype(jnp.float32)
        w_ref[2:3, :] = jnp.ones((1, _BV), jnp.float32)

    @pl.when(v < nsteps - 2)
    def _():
        t = t_ref[...]
        tnorm = jnp.sqrt(jnp.sum(t * t, axis=1, keepdims=True))
        tn = (t / jnp.maximum(tnorm, 1e-12)).astype(jnp.bfloat16)
        sim_ref[v % 3] = jax.lax.dot_general(
            tn, pn_ref[...], (((1,), (1,)), ((), ())),
            preferred_element_type=jnp.float32)

    @pl.when((v > 0) & (v < nsteps - 1))
    def _():
        mblk_ref[(v - 1) % 2] = jnp.max(sim_ref[(v - 1) % 3], axis=0,
                                        keepdims=True)

    @pl.when(v > 1)
    def _():
        b = v - 2
        sim = sim_ref[b % 3]
        m_c = mblk_ref[b % 2]
        eq01 = jnp.where(sim == m_c, 1.0, 0.0).astype(jnp.bfloat16)
        ext = jax.lax.dot_general(
            w_ref[...], eq01, (((1,), (0,)), ((), ())),
            preferred_element_type=jnp.float32)
        a_blk_ref[...] = (16.0 * ext[0:1, :] + ext[1:2, :]).astype(jnp.int32)
        n_eq = jnp.sum(ext[2:3, :])

        @pl.when(n_eq != float(sim.shape[1]))
        def _():
            iota_s = jax.lax.broadcasted_iota(jnp.int32, sim.shape, 0)
            cand = jnp.where(sim == m_c, iota_s, _BV)
            a_blk_ref[...] = jnp.min(cand, axis=0, keepdims=True)

        a_c = a_blk_ref[...] + b * _BV

        @pl.when(v == 2)
        def _():
            m_ref[...] = m_c
            a_ref[...] = a_c

        @pl.when(v == _HALF_MERGE)
        def _():
            m_ref[...] = m_ref[...].astype(jnp.bfloat16).astype(jnp.float32)

        @pl.when(v > 2)
        def _():
            take = m_c > m_ref[...]
            m_ref[...] = jnp.where(take, m_c, m_ref[...])
            a_ref[...] = jnp.where(take, a_c, a_ref[...])

        @pl.when(v == nsteps - 1)
        def _():
            o_ref[...] = a_ref[...]


def kernel(projections, table, top_k=1):
    b, s, e = projections.shape
    rows = b * s
    p2 = projections.reshape(rows, e)
    out = pl.pallas_call(
        _body,
        grid=(_NBLK + 2,),
        in_specs=[
            pl.BlockSpec((rows, e), lambda v: (0, 0)),
            pl.BlockSpec((_BV, e), lambda v: (jnp.minimum(v, _NBLK - 1), 0)),
        ],
        out_specs=pl.BlockSpec((1, rows), lambda v: (0, 0)),
        out_shape=jax.ShapeDtypeStruct((1, rows), jnp.int32),
        compiler_params=pltpu.CompilerParams(
            vmem_limit_bytes=100 * 1024 * 1024),
        scratch_shapes=[
            pltpu.VMEM((rows, _EMBED), jnp.bfloat16),
            pltpu.VMEM((3, _BV, rows), jnp.float32),
            pltpu.VMEM((2, 1, rows), jnp.float32),
            pltpu.VMEM((3, _BV), jnp.float32),
            pltpu.VMEM((1, rows), jnp.int32),
            pltpu.VMEM((1, rows), jnp.float32),
            pltpu.VMEM((1, rows), jnp.int32),
        ],
    )(p2, table)
    return out.reshape(b, s)


# speedup vs baseline: 1.1527x; 1.1527x over previous
import jax
import jax.numpy as jnp
from jax.experimental import pallas as pl
from jax.experimental.pallas import tpu as pltpu

_VOCAB = 8192
_EMBED = 256
_BV = 1024
_NBLK = _VOCAB // _BV
_HALF_STEP = _NBLK // 2


def _body(p_ref, t_ref, o_ref, pn_ref, sim_ref, w_ref, a_blk_ref,
          m_ref, a_ref):
    v = pl.program_id(0)
    nsteps = pl.num_programs(0)

    @pl.when(v == 0)
    def _():
        p = p_ref[...]
        pnorm = jnp.sqrt(jnp.sum(p * p, axis=1, keepdims=True))
        pn_ref[...] = (p / jnp.maximum(pnorm, 1e-12)).astype(jnp.bfloat16)
        s = jax.lax.broadcasted_iota(jnp.int32, (1, _BV), 1)
        w_ref[0:1, :] = (s // 16).astype(jnp.float32)
        w_ref[1:2, :] = (s % 16).astype(jnp.float32)
        w_ref[2:3, :] = jnp.ones((1, _BV), jnp.float32)

    @pl.when(v < nsteps - 1)
    def _():
        t = t_ref[...]
        tnorm = jnp.sqrt(jnp.sum(t * t, axis=1, keepdims=True))
        tn = (t / jnp.maximum(tnorm, 1e-12)).astype(jnp.bfloat16)
        sim_ref[v % 2] = jax.lax.dot_general(
            tn, pn_ref[...], (((1,), (1,)), ((), ())),
            preferred_element_type=jnp.float32)

    @pl.when(v > 0)
    def _():
        sim = sim_ref[(v - 1) % 2]
        m_c = jnp.max(sim, axis=0, keepdims=True)
        eq01 = jnp.where(sim == m_c, 1.0, 0.0).astype(jnp.bfloat16)
        ext = jax.lax.dot_general(
            w_ref[...], eq01, (((1,), (0,)), ((), ())),
            preferred_element_type=jnp.float32)
        a_blk_ref[...] = (16.0 * ext[0:1, :] + ext[1:2, :]).astype(jnp.int32)
        n_eq = jnp.sum(ext[2:3, :])

        @pl.when(n_eq != float(sim.shape[1]))
        def _():
            iota_s = jax.lax.broadcasted_iota(jnp.int32, sim.shape, 0)
            cand = jnp.where(sim == m_c, iota_s, _BV)
            a_blk_ref[...] = jnp.min(cand, axis=0, keepdims=True)

        a_c = a_blk_ref[...] + (v - 1) * _BV

        @pl.when(v == 1)
        def _():
            m_ref[...] = m_c
            a_ref[...] = a_c

        @pl.when(v == _HALF_STEP + 1)
        def _():
            m_ref[...] = m_ref[...].astype(jnp.bfloat16).astype(jnp.float32)

        @pl.when(v > 1)
        def _():
            take = m_c > m_ref[...]
            m_ref[...] = jnp.where(take, m_c, m_ref[...])
            a_ref[...] = jnp.where(take, a_c, a_ref[...])

        @pl.when(v == nsteps - 1)
        def _():
            o_ref[...] = a_ref[...]


def kernel(projections, table, top_k=1):
    b, s, e = projections.shape
    rows = b * s
    p2 = projections.reshape(rows, e)
    out = pl.pallas_call(
        _body,
        grid=(_NBLK + 1,),
        in_specs=[
            pl.BlockSpec((rows, e), lambda v: (0, 0)),
            pl.BlockSpec((_BV, e), lambda v: (jnp.minimum(v, _NBLK - 1), 0)),
        ],
        out_specs=pl.BlockSpec((1, rows), lambda v: (0, 0)),
        out_shape=jax.ShapeDtypeStruct((1, rows), jnp.int32),
        compiler_params=pltpu.CompilerParams(
            vmem_limit_bytes=100 * 1024 * 1024),
        scratch_shapes=[
            pltpu.VMEM((rows, _EMBED), jnp.bfloat16),
            pltpu.VMEM((2, _BV, rows), jnp.float32),
            pltpu.VMEM((3, _BV), jnp.float32),
            pltpu.VMEM((1, rows), jnp.int32),
            pltpu.VMEM((1, rows), jnp.float32),
            pltpu.VMEM((1, rows), jnp.int32),
        ],
    )(p2, table)
    return out.reshape(b, s)
